# baseline (speedup 1.0000x reference)
import functools

import jax
import jax.numpy as jnp
from jax import lax
from jax.experimental import pallas as pl
from jax.experimental.pallas import tpu as pltpu
from jax.experimental.pallas import tpu_sc as plsc

NC = 2
NS = 16
NW = NC * NS
L = 16

DIM = 64
CHUNK = 128


def _make_sc_kernel(batch: int):
    b_per_w = batch // NW
    n_chunks = b_per_w // CHUNK
    blocks_per_chunk = CHUNK // L

    mesh = plsc.VectorSubcoreMesh(core_axis_name="c", subcore_axis_name="s")

    @functools.partial(
        pl.kernel,
        out_type=jax.ShapeDtypeStruct((NW, b_per_w), jnp.float32),
        mesh=mesh,
        compiler_params=pltpu.CompilerParams(
            needs_layout_passes=False, use_tc_tiling_on_sc=False),
        scratch_types=[
            pltpu.VMEM((n_chunks, CHUNK), jnp.int32),
            pltpu.VMEM((n_chunks, CHUNK), jnp.int32),
            pltpu.VMEM((b_per_w, DIM), jnp.float32),
            pltpu.VMEM((b_per_w, DIM), jnp.float32),
            pltpu.VMEM((b_per_w,), jnp.float32),
            pltpu.SemaphoreType.DMA((n_chunks,)),
            pltpu.SemaphoreType.DMA((n_chunks,)),
        ],
    )
    def sc_kernel(u_hbm, i_hbm, uemb_hbm, iemb_hbm, out_hbm,
                  uidx_v, iidx_v, urows_v, irows_v, scores_v, usem, isem):
        wid = lax.axis_index("s") * NC + lax.axis_index("c")

        pltpu.sync_copy(u_hbm.at[wid], uidx_v)
        pltpu.sync_copy(i_hbm.at[wid], iidx_v)

        u_copies = []
        i_copies = []
        for c in range(n_chunks):
            dst_u = urows_v.at[pl.ds(c * CHUNK, CHUNK), :]
            dst_i = irows_v.at[pl.ds(c * CHUNK, CHUNK), :]
            u_copies.append(
                pltpu.async_copy(uemb_hbm.at[uidx_v.at[c]], dst_u, usem.at[c]))
            i_copies.append(
                pltpu.async_copy(iemb_hbm.at[iidx_v.at[c]], dst_i, isem.at[c]))

        iota = lax.iota(jnp.int32, L)

        def block_body(blk, c_base):
            row_idx = c_base + blk * L + iota
            acc0 = jnp.zeros((L,), jnp.float32)
            acc1 = jnp.zeros((L,), jnp.float32)
            acc2 = jnp.zeros((L,), jnp.float32)
            acc3 = jnp.zeros((L,), jnp.float32)
            accs = [acc0, acc1, acc2, acc3]
            for d in range(DIM):
                col = jnp.full((L,), d, jnp.int32)
                cu = plsc.load_gather(urows_v, [row_idx, col])
                ci = plsc.load_gather(irows_v, [row_idx, col])
                accs[d % 4] = accs[d % 4] + cu * ci
            score = (accs[0] + accs[1]) + (accs[2] + accs[3])
            scores_v[pl.ds(c_base + blk * L, L)] = score
            return c_base

        for c in range(n_chunks):
            u_copies[c].wait()
            i_copies[c].wait()
            lax.fori_loop(0, blocks_per_chunk, block_body, c * CHUNK,
                          unroll=False)

        pltpu.sync_copy(scores_v, out_hbm.at[wid])

    return sc_kernel


@jax.jit
def kernel(u, i, user_emb, item_emb):
    batch = u.shape[0]
    u_r = u.reshape(NW, batch // NW // CHUNK, CHUNK)
    i_r = i.reshape(NW, batch // NW // CHUNK, CHUNK)
    scores = _make_sc_kernel(batch)(u_r, i_r, user_emb, item_emb)
    return scores.reshape(batch)


# speedup vs baseline: 6.2373x; 6.2373x over previous
import functools

import jax
import jax.numpy as jnp
from jax import lax
from jax.experimental import pallas as pl
from jax.experimental.pallas import tpu as pltpu
from jax.experimental.pallas import tpu_sc as plsc

NC = 2
NS = 16
NW = NC * NS
L = 16

DIM = 64
G = 16


def _make_sc_kernel(batch: int, n_rows: int):
    b_per_w = batch // NW
    n_groups = b_per_w // G

    mesh = plsc.VectorSubcoreMesh(core_axis_name="c", subcore_axis_name="s")

    @functools.partial(
        pl.kernel,
        out_type=jax.ShapeDtypeStruct((NW, b_per_w), jnp.float32),
        mesh=mesh,
        compiler_params=pltpu.CompilerParams(
            needs_layout_passes=False, use_tc_tiling_on_sc=True),
        scratch_types=[
            pltpu.VMEM((b_per_w,), jnp.int32),
            pltpu.VMEM((b_per_w,), jnp.int32),
            pltpu.VMEM((2, G // 8, 8, 8, 128), jnp.float32),
            pltpu.VMEM((2, G // 8, 8, 8, 128), jnp.float32),
            pltpu.VMEM((b_per_w,), jnp.float32),
            pltpu.SemaphoreType.DMA((2,)),
            pltpu.SemaphoreType.DMA((2,)),
        ],
    )
    def sc_kernel(u_hbm, i_hbm, ut_hbm, it_hbm, out_hbm,
                  uidx_v, iidx_v, ublk_v, iblk_v, scores_v, usem, isem):
        wid = lax.axis_index("s") * NC + lax.axis_index("c")

        pltpu.sync_copy(u_hbm.at[wid], uidx_v)
        pltpu.sync_copy(i_hbm.at[wid], iidx_v)

        iota = lax.iota(jnp.int32, L)
        lane_mask = iota == (L - 1)
        a_idx = [((16 * k + jnp.arange(L)) // 8).astype(jnp.int32)
                 for k in range(4)]
        s_idx = [((16 * k + jnp.arange(L)) % 8).astype(jnp.int32)
                 for k in range(4)]
        a_idx = [jnp.asarray(a) for a in a_idx]
        s_idx = [jnp.asarray(s) for s in s_idx]

        def issue_group(g, parity):
            base = pl.multiple_of(g * G, G)
            uvec = uidx_v[pl.ds(base, G)]
            ivec = iidx_v[pl.ds(base, G)]
            for j in range(G):
                off_u = pl.multiple_of(uvec[j] & ~(L - 1), L)
                off_i = pl.multiple_of(ivec[j] & ~(L - 1), L)
                slot = pl.ds(L * (j % 8), L)
                pltpu.async_copy(
                    ut_hbm.at[:, :, pl.ds(off_u, L)],
                    ublk_v.at[parity, j // 8, :, :, slot], usem.at[parity])
                pltpu.async_copy(
                    it_hbm.at[:, :, pl.ds(off_i, L)],
                    iblk_v.at[parity, j // 8, :, :, slot], isem.at[parity])

        def drain_group(parity):
            for j in range(G):
                slot = pl.ds(L * (j % 8), L)
                pltpu.make_async_copy(
                    ut_hbm.at[:, :, pl.ds(0, L)],
                    ublk_v.at[parity, j // 8, :, :, slot],
                    usem.at[parity]).wait()
                pltpu.make_async_copy(
                    it_hbm.at[:, :, pl.ds(0, L)],
                    iblk_v.at[parity, j // 8, :, :, slot],
                    isem.at[parity]).wait()

        issue_group(0, 0)

        def body(g, carry):
            p = lax.rem(g, 2)

            @pl.when(g + 1 < n_groups)
            def _():
                issue_group(g + 1, lax.rem(g + 1, 2))

            drain_group(p)

            base = pl.multiple_of(g * G, G)
            uvec = uidx_v[pl.ds(base, G)]
            ivec = iidx_v[pl.ds(base, G)]
            cu_all = uvec & (L - 1)
            ci_all = ivec & (L - 1)
            for j in range(G):
                cu = lax.broadcast(cu_all[j] + L * (j % 8), (L,))
                ci = lax.broadcast(ci_all[j] + L * (j % 8), (L,))
                ublk = ublk_v.at[p, j // 8]
                iblk = iblk_v.at[p, j // 8]
                prods = []
                for k in range(4):
                    eu = plsc.load_gather(ublk, [a_idx[k], s_idx[k], cu])
                    ei = plsc.load_gather(iblk, [a_idx[k], s_idx[k], ci])
                    prods.append(eu * ei)
                acc = (prods[0] + prods[1]) + (prods[2] + prods[3])
                total = plsc.cumsum(acc)
                pos = lax.broadcast(g * G + j, (L,))
                plsc.store_scatter(scores_v, [pos], total, mask=lane_mask)
            return carry

        lax.fori_loop(0, n_groups, body, 0, unroll=False)

        pltpu.sync_copy(scores_v, out_hbm.at[wid])

    return sc_kernel


@jax.jit
def kernel(u, i, user_emb, item_emb):
    batch = u.shape[0]
    n_rows, dim = user_emb.shape
    ut3 = user_emb.T.reshape(8, dim // 8, n_rows)
    it3 = item_emb.T.reshape(8, dim // 8, n_rows)
    u_r = u.reshape(NW, batch // NW)
    i_r = i.reshape(NW, batch // NW)
    scores = _make_sc_kernel(batch, n_rows)(u_r, i_r, ut3, it3)
    return scores.reshape(batch)
